# Initial kernel scaffold; baseline (speedup 1.0000x reference)
import jax
import jax.numpy as jnp
from jax.experimental import pallas as pl


def kernel(x, edge_index, W1, a_src1, a_dst1, b1, W2, a_src2, a_dst2, b2):
    raise NotImplementedError("write your pallas kernel here")


# speedup vs baseline: 1.1093x; 1.1093x over previous
import functools

import jax
import jax.numpy as jnp
from jax.experimental import pallas as pl
from jax.experimental.pallas import tpu as pltpu

N = 10000
E = 320000
D_IN = 128
HID = 128
OUT = 128
H1 = 8

_BLK = 1000


def _mm_kernel(x_ref, w_ref, o_ref):
    o_ref[...] = jnp.dot(x_ref[...], w_ref[...],
                         preferred_element_type=jnp.float32)


def _matmul(x, w):
    n, k = x.shape
    k2, m = w.shape
    grid = (n // _BLK,)
    return pl.pallas_call(
        _mm_kernel,
        grid=grid,
        in_specs=[
            pl.BlockSpec((_BLK, k), lambda i: (i, 0)),
            pl.BlockSpec((k, m), lambda i: (0, 0)),
        ],
        out_specs=pl.BlockSpec((_BLK, m), lambda i: (i, 0)),
        out_shape=jax.ShapeDtypeStruct((n, m), jnp.float32),
    )(x, w)


def _gat_layer(x, src, dst, valid, W, a_src, a_dst, bias, heads, ch, concat):
    n = x.shape[0]
    h = _matmul(x, W).reshape(n, heads, ch)
    alpha_s = jnp.sum(h * a_src[None, :, :], axis=-1)
    alpha_d = jnp.sum(h * a_dst[None, :, :], axis=-1)
    e = jax.nn.leaky_relu(alpha_s[src] + alpha_d[dst], negative_slope=0.2)
    ex = jnp.where(valid[:, None], jnp.exp(e), 0.0)
    den = jax.ops.segment_sum(ex, dst, num_segments=n)
    alpha = ex / (den[dst] + 1e-16)
    out = jax.ops.segment_sum(h[src] * alpha[:, :, None], dst, num_segments=n)
    if concat:
        out = out.reshape(n, heads * ch)
    else:
        out = out.mean(axis=1)
    return out + bias


def kernel(x, edge_index, W1, a_src1, a_dst1, b1, W2, a_src2, a_dst2, b2):
    src0, dst0 = edge_index[0], edge_index[1]
    loop = jnp.arange(N, dtype=src0.dtype)
    src = jnp.concatenate([src0, loop])
    dst = jnp.concatenate([dst0, loop])
    valid = jnp.concatenate([src0 != dst0, jnp.ones((N,), dtype=bool)])
    h = _gat_layer(x, src, dst, valid, W1, a_src1, a_dst1, b1, H1, HID, True)
    h = jax.nn.elu(h)
    out = _gat_layer(h, src, dst, valid, W2, a_src2, a_dst2, b2, 1, OUT, False)
    return out


# speedup vs baseline: 3.8084x; 3.4331x over previous
import functools

import jax
import jax.numpy as jnp
from jax import lax
from jax.experimental import pallas as pl
from jax.experimental.pallas import tpu as pltpu
from jax.experimental.pallas import tpu_sc as plsc

N = 10000
N_PAD = 10240
E = 320000
D_IN = 128
HID = 128
OUT = 128
H1 = 8

NC, NS, NW = 2, 16, 32
RPT = 88
PT = RPT * 128
E_PAD = NW * PT
NROW = NW * RPT
NG = PT // 16

_MESH = dict(core_axis_name="c", subcore_axis_name="s",
             num_cores=NC, num_subcores=NS)


def _tc1_body(x_ref, w_ref, as_ref, ad_ref, h_ref, oas_ref, oad_ref):
    h = jnp.dot(x_ref[...], w_ref[...], preferred_element_type=jnp.float32)
    h_ref[0, 0] = h[:, :64]
    h_ref[1, 0] = h[:, 64:]
    hh = pl.ds(pl.program_id(0), 1)
    oas_ref[0] = jnp.sum(h * as_ref[hh], axis=1, keepdims=True)
    oad_ref[0] = jnp.sum(h * ad_ref[hh], axis=1, keepdims=True)


def _tc1(x, W1, a_src1, a_dst1):
    blk = 1000
    return pl.pallas_call(
        _tc1_body,
        grid=(H1, N // blk),
        in_specs=[
            pl.BlockSpec((blk, D_IN), lambda h, i: (i, 0)),
            pl.BlockSpec((D_IN, HID), lambda h, i: (0, h)),
            pl.BlockSpec((H1, HID), lambda h, i: (0, 0)),
            pl.BlockSpec((H1, HID), lambda h, i: (0, 0)),
        ],
        out_specs=[
            pl.BlockSpec((2, 1, blk, 64), lambda h, i: (0, h, i, 0)),
            pl.BlockSpec((1, blk, 1), lambda h, i: (h, i, 0)),
            pl.BlockSpec((1, blk, 1), lambda h, i: (h, i, 0)),
        ],
        out_shape=[
            jax.ShapeDtypeStruct((2, H1, N, 64), jnp.float32),
            jax.ShapeDtypeStruct((H1, N, 1), jnp.float32),
            jax.ShapeDtypeStruct((H1, N, 1), jnp.float32),
        ],
    )(x, W1, a_src1, a_dst1)


def _tc2_body(*refs):
    p_refs = refs[:2 * H1]
    b1_ref, w_ref, as_ref, ad_ref = refs[2 * H1:2 * H1 + 4]
    h_ref, oas_ref, oad_ref = refs[2 * H1 + 4:]
    parts = []
    for h in range(H1):
        v = jnp.concatenate(
            [p_refs[2 * h][0, 0] + p_refs[2 * h][1, 0],
             p_refs[2 * h + 1][0, 0] + p_refs[2 * h + 1][1, 0]], axis=1)
        v = v + b1_ref[h][None, :]
        parts.append(jnp.where(v > 0, v, jnp.exp(v) - jnp.float32(1.0)))
    x2 = jnp.concatenate(parts, axis=1)
    h2 = jnp.dot(x2, w_ref[...], preferred_element_type=jnp.float32)
    h_ref[0] = h2[:, :64]
    h_ref[1] = h2[:, 64:]
    oas_ref[...] = jnp.sum(h2 * as_ref[...], axis=1, keepdims=True)
    oad_ref[...] = jnp.sum(h2 * ad_ref[...], axis=1, keepdims=True)


def _tc2(p1, b1, W2, a_src2, a_dst2):
    blk = 1000
    in_specs = [
        pl.BlockSpec((NC, 1, blk, 64),
                     functools.partial(lambda i, hh: (0, hh, i, 0), hh=h))
        for h in range(2 * H1)
    ] + [
        pl.BlockSpec((H1, HID), lambda i: (0, 0)),
        pl.BlockSpec((H1 * HID, OUT), lambda i: (0, 0)),
        pl.BlockSpec((1, OUT), lambda i: (0, 0)),
        pl.BlockSpec((1, OUT), lambda i: (0, 0)),
    ]
    return pl.pallas_call(
        _tc2_body,
        grid=(N // blk,),
        in_specs=in_specs,
        out_specs=[
            pl.BlockSpec((2, blk, 64), lambda i: (0, i, 0)),
            pl.BlockSpec((blk, 1), lambda i: (i, 0)),
            pl.BlockSpec((blk, 1), lambda i: (i, 0)),
        ],
        out_shape=[
            jax.ShapeDtypeStruct((2, N, 64), jnp.float32),
            jax.ShapeDtypeStruct((N, 1), jnp.float32),
            jax.ShapeDtypeStruct((N, 1), jnp.float32),
        ],
    )(*([p1] * (2 * H1)), b1.reshape(H1, HID), W2, a_src2, a_dst2)


def _tc3_body(p_ref, b_ref, o_ref):
    t = p_ref[0] + p_ref[1]
    o_ref[...] = jnp.concatenate([t[0], t[1]], axis=1) + b_ref[...]


def _tc3(p2, b2):
    blk = 1000
    return pl.pallas_call(
        _tc3_body,
        grid=(N // blk,),
        in_specs=[
            pl.BlockSpec((NC, 2, blk, 64), lambda i: (0, 0, i, 0)),
            pl.BlockSpec((1, OUT), lambda i: (0, 0)),
        ],
        out_specs=pl.BlockSpec((blk, OUT), lambda i: (i, 0)),
        out_shape=jax.ShapeDtypeStruct((N, OUT), jnp.float32),
    )(p2, b2.reshape(1, OUT))


def _make_phase_a(H):
    def body(edata, as_hbm, ad_hbm, ex_hbm, den_hbm,
             as_t, ad_t, sbuf, dbuf, vbuf, exbuf, idxbuf, zbuf, den_sh, sem):
        c = lax.axis_index("c")
        s = lax.axis_index("s")
        row0 = (c * NS + s) * RPT

        def zfill(i, _):
            zbuf[pl.ds(i * 16, 16)] = jnp.zeros((16,), jnp.float32)
            return ()
        lax.fori_loop(0, 40, zfill, (), unroll=False)
        zlen = H * N_PAD // NS
        for i in range(zlen // 640):
            pltpu.sync_copy(zbuf,
                            den_sh.at[pl.ds(s * zlen + i * 640, 640)])
        plsc.subcore_barrier()

        pltpu.sync_copy(edata.at[0].at[pl.ds(row0, RPT)], sbuf)
        pltpu.sync_copy(edata.at[1].at[pl.ds(row0, RPT)], dbuf)
        pltpu.sync_copy(edata.at[2].at[pl.ds(row0, RPT)], vbuf)

        for h in range(H):
            pltpu.sync_copy(as_hbm.at[h], as_t)
            pltpu.sync_copy(ad_hbm.at[h], ad_t)

            def group(g, _):
                j = g // 8
                r = (g % 8) * 16
                sv = sbuf[j, pl.ds(r, 16)]
                dv = dbuf[j, pl.ds(r, 16)]
                vv = vbuf[j, pl.ds(r, 16)]
                a = plsc.load_gather(as_t, [sv])
                b = plsc.load_gather(ad_t, [dv])
                t = a + b
                t = jnp.where(t >= 0, t, t * jnp.float32(0.2))
                exv = jnp.exp(t) * vv.astype(jnp.float32)
                exbuf[j, pl.ds(r, 16)] = exv
                idxbuf[j, pl.ds(r, 16)] = dv + jnp.int32(h * N_PAD)
                return ()

            lax.fori_loop(0, NG, group, (), unroll=False)
            pltpu.sync_copy(exbuf, ex_hbm.at[h].at[pl.ds(row0, RPT)])

            def dscat(j, _):
                pltpu.async_copy(exbuf.at[j], den_sh.at[idxbuf.at[j]], sem,
                                 add=True)
                return ()

            lax.fori_loop(0, RPT, dscat, (), unroll=False)
            pltpu.make_async_copy(ex_hbm.at[h].at[pl.ds(0, RPT)],
                                  exbuf, sem).wait()

        plsc.subcore_barrier()
        zlen = H * N_PAD // NS
        pltpu.sync_copy(den_sh.at[pl.ds(s * zlen, zlen)],
                        den_hbm.at[c].at[pl.ds(s * zlen, zlen)])

    mesh = plsc.VectorSubcoreMesh(**_MESH)
    return functools.partial(
        pl.kernel, body,
        out_type=[
            jax.ShapeDtypeStruct((H, NROW, 128), jnp.float32),
            jax.ShapeDtypeStruct((NC, H * N_PAD), jnp.float32),
        ],
        mesh=mesh,
        compiler_params=pltpu.CompilerParams(needs_layout_passes=False, use_tc_tiling_on_sc=False),
        scratch_types=[
            pltpu.VMEM((N,), jnp.float32),
            pltpu.VMEM((N,), jnp.float32),
            pltpu.VMEM((RPT, 128), jnp.int32),
            pltpu.VMEM((RPT, 128), jnp.int32),
            pltpu.VMEM((RPT, 128), jnp.int32),
            pltpu.VMEM((RPT, 128), jnp.float32),
            pltpu.VMEM((RPT, 128), jnp.int32),
            pltpu.VMEM((640,), jnp.float32),
            pltpu.VMEM_SHARED((H * N_PAD,), jnp.float32),
            pltpu.SemaphoreType.DMA,
        ])()


def _make_phase_b(H):
    def body(edata, ex_hbm, den_hbm, feat, out_hbm,
             den_t, dtmp, exb, albuf, sidx, didx, rows, acc_sh,
             semg0, semg1, sems):
        c = lax.axis_index("c")
        s = lax.axis_index("s")
        row0 = (c * NS + s) * RPT

        pltpu.sync_copy(edata.at[0].at[pl.ds(row0, RPT)], sidx)
        pltpu.sync_copy(edata.at[1].at[pl.ds(row0, RPT)], didx)

        def shift_sidx(delta):
            def sh(g, _):
                j = g // 8
                r = (g % 8) * 16
                sidx[j, pl.ds(r, 16)] = (sidx[j, pl.ds(r, 16)]
                                         + jnp.int32(delta))
                return ()
            lax.fori_loop(0, NG, sh, (), unroll=False)

        for h in range(H):
            pltpu.sync_copy(den_hbm.at[0].at[pl.ds(h * N_PAD, N_PAD)], den_t)
            pltpu.sync_copy(den_hbm.at[1].at[pl.ds(h * N_PAD, N_PAD)], dtmp)

            def dsum(i, _):
                v = den_t[pl.ds(i * 16, 16)] + dtmp[pl.ds(i * 16, 16)]
                den_t[pl.ds(i * 16, 16)] = v + jnp.float32(1e-16)
                return ()
            lax.fori_loop(0, N_PAD // 16, dsum, (), unroll=False)

            pltpu.sync_copy(ex_hbm.at[h].at[pl.ds(row0, RPT)], exb)

            def agroup(g, _):
                j = g // 8
                r = (g % 8) * 16
                dv = didx[j, pl.ds(r, 16)]
                dnv = plsc.load_gather(den_t, [dv])
                albuf[j, pl.ds(r, 16)] = exb[j, pl.ds(r, 16)] / dnv
                return ()
            lax.fori_loop(0, NG, agroup, (), unroll=False)

            for f in range(2):
                if h == 0 and f == 0:
                    pass
                elif f == 1:
                    shift_sidx(H * N)
                else:
                    shift_sidx(N - H * N)

                def zrows(i, _):
                    r = rows.at[0].at[i]
                    for q in range(4):
                        r[pl.ds(q * 16, 16)] = jnp.zeros((16,), jnp.float32)
                    return ()
                lax.fori_loop(0, 128, zrows, (), unroll=False)
                for i in range(5):
                    pltpu.sync_copy(rows.at[0],
                                    acc_sh.at[pl.ds(s * 640 + i * 128, 128)])
                plsc.subcore_barrier()

                def gather(k, b, semg):
                    pltpu.async_copy(feat.at[sidx.at[k]], rows.at[b], semg)

                def drain_gather(b, semg):
                    pltpu.make_async_copy(feat.at[pl.ds(0, 128)],
                                          rows.at[b], semg).wait()

                def scale_and_scatter(k, b):
                    def scale(g, _):
                        alv = albuf[k, pl.ds(g * 16, 16)]
                        for l in range(16):
                            av = jnp.full((16,), alv[l], jnp.float32)
                            r = rows.at[b].at[g * 16 + l]
                            for q in range(4):
                                r[pl.ds(q * 16, 16)] = (r[pl.ds(q * 16, 16)]
                                                        * av)
                        return ()
                    lax.fori_loop(0, 8, scale, (), unroll=False)
                    pltpu.sync_copy(rows.at[b], acc_sh.at[didx.at[k]],
                                    add=True)

                gather(0, 0, semg0)

                def chunk2(m, _):
                    k0 = m * 2
                    gather(k0 + 1, 1, semg1)
                    drain_gather(0, semg0)
                    scale_and_scatter(k0, 0)

                    @pl.when(m + 1 < RPT // 2)
                    def _():
                        gather(k0 + 2, 0, semg0)

                    drain_gather(1, semg1)
                    scale_and_scatter(k0 + 1, 1)
                    return ()

                lax.fori_loop(0, RPT // 2, chunk2, (), unroll=False)
                plsc.subcore_barrier()
                pltpu.sync_copy(
                    acc_sh.at[pl.ds(s * 640, 640)],
                    out_hbm.at[c].at[2 * h + f].at[pl.ds(s * 640, 640)])

    mesh = plsc.VectorSubcoreMesh(**_MESH)
    return functools.partial(
        pl.kernel, body,
        out_type=jax.ShapeDtypeStruct((NC, 2 * H, N_PAD, 64), jnp.float32),
        mesh=mesh,
        compiler_params=pltpu.CompilerParams(needs_layout_passes=False, use_tc_tiling_on_sc=False),
        scratch_types=[
            pltpu.VMEM((N_PAD,), jnp.float32),
            pltpu.VMEM((N_PAD,), jnp.float32),
            pltpu.VMEM((RPT, 128), jnp.float32),
            pltpu.VMEM((RPT, 128), jnp.float32),
            pltpu.VMEM((RPT, 128), jnp.int32),
            pltpu.VMEM((RPT, 128), jnp.int32),
            pltpu.VMEM((2, 128, 64), jnp.float32),
            pltpu.VMEM_SHARED((N_PAD, 64), jnp.float32),
            pltpu.SemaphoreType.DMA,
            pltpu.SemaphoreType.DMA,
            pltpu.SemaphoreType.DMA,
        ])()


_phase_a1 = _make_phase_a(H1)
_phase_a2 = _make_phase_a(1)
_phase_b1 = _make_phase_b(H1)
_phase_b2 = _make_phase_b(1)


def kernel(x, edge_index, W1, a_src1, a_dst1, b1, W2, a_src2, a_dst2, b2):
    src0, dst0 = edge_index[0], edge_index[1]
    loop = jnp.arange(N, dtype=jnp.int32)
    pad = E_PAD - (E + N)
    src = jnp.concatenate([src0.astype(jnp.int32), loop,
                           jnp.zeros((pad,), jnp.int32)])
    dst = jnp.concatenate([dst0.astype(jnp.int32), loop,
                           jnp.zeros((pad,), jnp.int32)])
    valid = jnp.concatenate([
        (src0 != dst0).astype(jnp.int32),
        jnp.ones((N,), jnp.int32),
        jnp.zeros((pad,), jnp.int32)])
    edata = jnp.stack([src, dst, valid]).reshape(3, NROW, 128)

    h1, as1, ad1 = _tc1(x, W1, a_src1, a_dst1)
    ex1, den1 = _phase_a1(edata, as1.reshape(H1, N), ad1.reshape(H1, N))
    p1 = _phase_b1(edata, ex1, den1, h1.reshape(2 * H1 * N, 64))
    h2, as2, ad2 = _tc2(p1, b1, W2, a_src2, a_dst2)
    ex2, den2 = _phase_a2(edata, as2.reshape(1, N), ad2.reshape(1, N))
    p2 = _phase_b2(edata, ex2, den2, h2.reshape(2 * N, 64))
    return _tc3(p2, b2)


# speedup vs baseline: 4.1670x; 1.0942x over previous
import functools

import jax
import jax.numpy as jnp
from jax import lax
from jax.experimental import pallas as pl
from jax.experimental.pallas import tpu as pltpu
from jax.experimental.pallas import tpu_sc as plsc

N = 10000
N_PAD = 10240
E = 320000
D_IN = 128
HID = 128
OUT = 128
H1 = 8

NC, NS, NW = 2, 16, 32
RPT = 88
PT = RPT * 128
E_PAD = NW * PT
NROW = NW * RPT
NG = PT // 16

_MESH = dict(core_axis_name="c", subcore_axis_name="s",
             num_cores=NC, num_subcores=NS)
_SC_PARAMS = pltpu.CompilerParams(needs_layout_passes=False,
                                  use_tc_tiling_on_sc=False)


def _tc1_body(x_ref, w_ref, as_ref, ad_ref, h_ref, oas_ref, oad_ref):
    h = jnp.dot(x_ref[...], w_ref[...], preferred_element_type=jnp.float32)
    h_ref[0] = h
    hh = pl.ds(pl.program_id(0), 1)
    oas_ref[0] = jnp.sum(h * as_ref[hh], axis=1, keepdims=True)
    oad_ref[0] = jnp.sum(h * ad_ref[hh], axis=1, keepdims=True)


def _tc1(x, W1, a_src1, a_dst1):
    blk = 1000
    return pl.pallas_call(
        _tc1_body,
        grid=(H1, N // blk),
        in_specs=[
            pl.BlockSpec((blk, D_IN), lambda h, i: (i, 0)),
            pl.BlockSpec((D_IN, HID), lambda h, i: (0, h)),
            pl.BlockSpec((H1, HID), lambda h, i: (0, 0)),
            pl.BlockSpec((H1, HID), lambda h, i: (0, 0)),
        ],
        out_specs=[
            pl.BlockSpec((1, blk, HID), lambda h, i: (h, i, 0)),
            pl.BlockSpec((1, blk, 1), lambda h, i: (h, i, 0)),
            pl.BlockSpec((1, blk, 1), lambda h, i: (h, i, 0)),
        ],
        out_shape=[
            jax.ShapeDtypeStruct((H1, N, HID), jnp.float32),
            jax.ShapeDtypeStruct((H1, N, 1), jnp.float32),
            jax.ShapeDtypeStruct((H1, N, 1), jnp.float32),
        ],
    )(x, W1, a_src1, a_dst1)


def _tc2_body(*refs):
    p_refs = refs[:H1]
    d_refs = refs[H1:2 * H1]
    b1_ref, w_ref, as_ref, ad_ref = refs[2 * H1:2 * H1 + 4]
    h_ref, oas_ref, oad_ref = refs[2 * H1 + 4:]
    parts = []
    for h in range(H1):
        den = d_refs[h][0, 0] + d_refs[h][1, 0] + jnp.float32(1e-16)
        v = (p_refs[h][0, 0] + p_refs[h][1, 0]) / den + b1_ref[h][None, :]
        parts.append(jnp.where(v > 0, v, jnp.exp(v) - jnp.float32(1.0)))
    x2 = jnp.concatenate(parts, axis=1)
    h2 = jnp.dot(x2, w_ref[...], preferred_element_type=jnp.float32)
    h_ref[...] = h2
    oas_ref[...] = jnp.sum(h2 * as_ref[...], axis=1, keepdims=True)
    oad_ref[...] = jnp.sum(h2 * ad_ref[...], axis=1, keepdims=True)


def _tc2(p1, den1, b1, W2, a_src2, a_dst2):
    blk = 1000
    in_specs = [
        pl.BlockSpec((NC, 1, blk, HID),
                     functools.partial(lambda i, hh: (0, hh, i, 0), hh=h))
        for h in range(H1)
    ] + [
        pl.BlockSpec((NC, 1, blk, 1),
                     functools.partial(lambda i, hh: (0, hh, i, 0), hh=h))
        for h in range(H1)
    ] + [
        pl.BlockSpec((H1, HID), lambda i: (0, 0)),
        pl.BlockSpec((H1 * HID, OUT), lambda i: (0, 0)),
        pl.BlockSpec((1, OUT), lambda i: (0, 0)),
        pl.BlockSpec((1, OUT), lambda i: (0, 0)),
    ]
    return pl.pallas_call(
        _tc2_body,
        grid=(N // blk,),
        in_specs=in_specs,
        out_specs=[
            pl.BlockSpec((blk, OUT), lambda i: (i, 0)),
            pl.BlockSpec((blk, 1), lambda i: (i, 0)),
            pl.BlockSpec((blk, 1), lambda i: (i, 0)),
        ],
        out_shape=[
            jax.ShapeDtypeStruct((N, OUT), jnp.float32),
            jax.ShapeDtypeStruct((N, 1), jnp.float32),
            jax.ShapeDtypeStruct((N, 1), jnp.float32),
        ],
    )(*([p1] * H1), *([den1] * H1),
      b1.reshape(H1, HID), W2, a_src2, a_dst2)


def _tc3_body(p_ref, d_ref, b_ref, o_ref):
    den = d_ref[0, 0] + d_ref[1, 0] + jnp.float32(1e-16)
    o_ref[...] = (p_ref[0, 0] + p_ref[1, 0]) / den + b_ref[...]


def _tc3(p2, den2, b2):
    blk = 1000
    return pl.pallas_call(
        _tc3_body,
        grid=(N // blk,),
        in_specs=[
            pl.BlockSpec((NC, 1, blk, OUT), lambda i: (0, 0, i, 0)),
            pl.BlockSpec((NC, 1, blk, 1), lambda i: (0, 0, i, 0)),
            pl.BlockSpec((1, OUT), lambda i: (0, 0)),
        ],
        out_specs=pl.BlockSpec((blk, OUT), lambda i: (i, 0)),
        out_shape=jax.ShapeDtypeStruct((N, OUT), jnp.float32),
    )(p2, den2, b2.reshape(1, OUT))


def _make_phase_a(H):
    def body(edata, as_hbm, ad_hbm, ex_hbm, den_hbm,
             as_t, ad_t, sbuf, dbuf, vbuf, exbuf, idxbuf, zbuf, den_sh, sem):
        c = lax.axis_index("c")
        s = lax.axis_index("s")
        row0 = (c * NS + s) * RPT

        def zfill(i, _):
            zbuf[pl.ds(i * 16, 16)] = jnp.zeros((16,), jnp.float32)
            return ()
        lax.fori_loop(0, 40, zfill, (), unroll=False)
        zlen = H * N_PAD // NS
        for i in range(zlen // 640):
            pltpu.sync_copy(zbuf,
                            den_sh.at[pl.ds(s * zlen + i * 640, 640)])
        plsc.subcore_barrier()

        pltpu.sync_copy(edata.at[0].at[pl.ds(row0, RPT)], sbuf)
        pltpu.sync_copy(edata.at[1].at[pl.ds(row0, RPT)], dbuf)
        pltpu.sync_copy(edata.at[2].at[pl.ds(row0, RPT)], vbuf)

        for h in range(H):
            pltpu.sync_copy(as_hbm.at[h], as_t)
            pltpu.sync_copy(ad_hbm.at[h], ad_t)

            def group(g, _):
                j = g // 8
                r = (g % 8) * 16
                sv = sbuf[j, pl.ds(r, 16)]
                dv = dbuf[j, pl.ds(r, 16)]
                vv = vbuf[j, pl.ds(r, 16)]
                a = plsc.load_gather(as_t, [sv])
                b = plsc.load_gather(ad_t, [dv])
                t = a + b
                t = jnp.where(t >= 0, t, t * jnp.float32(0.2))
                exv = jnp.exp(t) * vv.astype(jnp.float32)
                exbuf[j, pl.ds(r, 16)] = exv
                idxbuf[j, pl.ds(r, 16)] = dv + jnp.int32(h * N_PAD)
                return ()

            lax.fori_loop(0, NG, group, (), unroll=False)
            pltpu.sync_copy(exbuf, ex_hbm.at[h].at[pl.ds(row0, RPT)])

            def dscat(j, _):
                pltpu.async_copy(exbuf.at[j], den_sh.at[idxbuf.at[j]], sem,
                                 add=True)
                return ()

            lax.fori_loop(0, RPT, dscat, (), unroll=False)
            pltpu.make_async_copy(ex_hbm.at[h].at[pl.ds(0, RPT)],
                                  exbuf, sem).wait()

        plsc.subcore_barrier()
        zlen = H * N_PAD // NS
        pltpu.sync_copy(den_sh.at[pl.ds(s * zlen, zlen)],
                        den_hbm.at[c].at[pl.ds(s * zlen, zlen)])

    mesh = plsc.VectorSubcoreMesh(**_MESH)
    return functools.partial(
        pl.kernel, body,
        out_type=[
            jax.ShapeDtypeStruct((H, NROW, 128), jnp.float32),
            jax.ShapeDtypeStruct((NC, H * N_PAD), jnp.float32),
        ],
        mesh=mesh,
        compiler_params=_SC_PARAMS,
        scratch_types=[
            pltpu.VMEM((N,), jnp.float32),
            pltpu.VMEM((N,), jnp.float32),
            pltpu.VMEM((RPT, 128), jnp.int32),
            pltpu.VMEM((RPT, 128), jnp.int32),
            pltpu.VMEM((RPT, 128), jnp.int32),
            pltpu.VMEM((RPT, 128), jnp.float32),
            pltpu.VMEM((RPT, 128), jnp.int32),
            pltpu.VMEM((640,), jnp.float32),
            pltpu.VMEM_SHARED((H * N_PAD,), jnp.float32),
            pltpu.SemaphoreType.DMA,
        ])()


def _make_phase_b(H):
    def body(edata, ex_hbm, feat, out_hbm,
             sidx, didx, exloc, rows, acc_sh, semg, seme, sems):
        c = lax.axis_index("c")
        s = lax.axis_index("s")
        row0 = (c * NS + s) * RPT

        pltpu.sync_copy(edata.at[0].at[pl.ds(row0, RPT)], sidx)
        pltpu.sync_copy(edata.at[1].at[pl.ds(row0, RPT)], didx)

        def vhead(h, _):
            @pl.when(h > 0)
            def _():
                def sh(g, _):
                    j = g // 8
                    r = (g % 8) * 16
                    sidx[j, pl.ds(r, 16)] = (sidx[j, pl.ds(r, 16)]
                                             + jnp.int32(N))
                    return ()
                lax.fori_loop(0, NG, sh, (), unroll=False)

            def zrows(i, _):
                r = rows.at[i]
                for q in range(8):
                    r[pl.ds(q * 16, 16)] = jnp.zeros((16,), jnp.float32)
                return ()
            lax.fori_loop(0, 128, zrows, (), unroll=False)
            for i in range(5):
                pltpu.sync_copy(rows,
                                acc_sh.at[pl.ds(s * 640 + i * 128, 128)])
            plsc.subcore_barrier()

            pltpu.async_copy(ex_hbm.at[h].at[row0], exloc.at[0], seme)

            def chunk(k, _):
                kb = k % 2
                pltpu.make_async_copy(ex_hbm.at[h].at[0],
                                      exloc.at[kb], seme).wait()

                @pl.when(k + 1 < RPT)
                def _():
                    pltpu.async_copy(ex_hbm.at[h].at[row0 + k + 1],
                                     exloc.at[1 - kb], seme)

                cp = pltpu.async_copy(feat.at[sidx.at[k]], rows, semg)
                cp.wait()

                def sc(g, _):
                    alv = exloc[kb, pl.ds(g * 16, 16)]
                    for l in range(16):
                        av = jnp.full((16,), alv[l], jnp.float32)
                        r = rows.at[g * 16 + l]
                        for q in range(8):
                            r[pl.ds(q * 16, 16)] = r[pl.ds(q * 16, 16)] * av
                    return ()
                lax.fori_loop(0, 8, sc, (), unroll=False)

                pltpu.sync_copy(rows, acc_sh.at[didx.at[k]], add=True)
                return ()

            lax.fori_loop(0, RPT, chunk, (), unroll=False)
            plsc.subcore_barrier()
            pltpu.sync_copy(
                acc_sh.at[pl.ds(s * 640, 640)],
                out_hbm.at[c].at[h].at[pl.ds(s * 640, 640)])
            return ()

        lax.fori_loop(0, H, vhead, (), unroll=False)

    mesh = plsc.VectorSubcoreMesh(**_MESH)
    return functools.partial(
        pl.kernel, body,
        out_type=jax.ShapeDtypeStruct((NC, H, N_PAD, HID), jnp.float32),
        mesh=mesh,
        compiler_params=_SC_PARAMS,
        scratch_types=[
            pltpu.VMEM((RPT, 128), jnp.int32),
            pltpu.VMEM((RPT, 128), jnp.int32),
            pltpu.VMEM((2, 128), jnp.float32),
            pltpu.VMEM((128, HID), jnp.float32),
            pltpu.VMEM_SHARED((N_PAD, HID), jnp.float32),
            pltpu.SemaphoreType.DMA,
            pltpu.SemaphoreType.DMA,
            pltpu.SemaphoreType.DMA,
        ])()


_phase_a1 = _make_phase_a(H1)
_phase_a2 = _make_phase_a(1)
_phase_b1 = _make_phase_b(H1)
_phase_b2 = _make_phase_b(1)


def kernel(x, edge_index, W1, a_src1, a_dst1, b1, W2, a_src2, a_dst2, b2):
    src0, dst0 = edge_index[0], edge_index[1]
    loop = jnp.arange(N, dtype=jnp.int32)
    pad = E_PAD - (E + N)
    src = jnp.concatenate([src0.astype(jnp.int32), loop,
                           jnp.zeros((pad,), jnp.int32)])
    dst = jnp.concatenate([dst0.astype(jnp.int32), loop,
                           jnp.zeros((pad,), jnp.int32)])
    valid = jnp.concatenate([
        (src0 != dst0).astype(jnp.int32),
        jnp.ones((N,), jnp.int32),
        jnp.zeros((pad,), jnp.int32)])
    edata = jnp.stack([src, dst, valid]).reshape(3, NROW, 128)

    h1, as1, ad1 = _tc1(x, W1, a_src1, a_dst1)
    ex1, den1 = _phase_a1(edata, as1.reshape(H1, N), ad1.reshape(H1, N))
    p1 = _phase_b1(edata, ex1, h1.reshape(H1 * N, HID))
    h2, as2, ad2 = _tc2(p1, den1.reshape(NC, H1, N_PAD, 1),
                        b1, W2, a_src2, a_dst2)
    ex2, den2 = _phase_a2(edata, as2.reshape(1, N), ad2.reshape(1, N))
    p2 = _phase_b2(edata, ex2, h2)
    return _tc3(p2, den2.reshape(NC, 1, N_PAD, 1), b2)
